# baseline (speedup 1.0000x reference)
import functools

import jax
import jax.numpy as jnp
from jax import lax
from jax.experimental import pallas as pl
from jax.experimental.pallas import tpu as pltpu
from jax.experimental.pallas import tpu_sc as plsc

_NBUF = 4
_LOOKAHEAD = 2
_CHUNK = 400
_SEQ_PAD = 56
_PAD_BLK = 800


def _pad_table(vectors):
    V, D = vectors.shape

    def body(i_ref, o_ref):
        o_ref[:, :D] = i_ref[...]
        o_ref[:, D:] = jnp.zeros((_PAD_BLK, 128 - D), jnp.float32)

    return pl.pallas_call(
        body,
        grid=(V // _PAD_BLK,),
        in_specs=[pl.BlockSpec((_PAD_BLK, D), lambda i: (i, 0))],
        out_specs=pl.BlockSpec((_PAD_BLK, 128), lambda i: (i, 0)),
        out_shape=jax.ShapeDtypeStruct((V, 128), jnp.float32),
    )(vectors)


def _gather_kernel(N, V2, n_workers):
    n_per_w = N // n_workers
    n_chunks = n_per_w // _CHUNK
    n_groups = n_chunks // _NBUF
    mesh = plsc.VectorSubcoreMesh(core_axis_name="c", subcore_axis_name="s")

    @functools.partial(
        pl.kernel,
        mesh=mesh,
        out_type=jax.ShapeDtypeStruct((2 * 16384 * _SEQ_PAD, 64), jnp.float32),
        scratch_types=[
            pltpu.VMEM((_NBUF, _CHUNK), jnp.int32),
            pltpu.VMEM((_NBUF, _CHUNK), jnp.int32),
            pltpu.VMEM((_NBUF, _CHUNK, 64), jnp.float32),
            pltpu.SemaphoreType.DMA((_NBUF,)),
            pltpu.SemaphoreType.DMA((_NBUF,)),
        ],
        compiler_params=pltpu.CompilerParams(use_tc_tiling_on_sc=False),
    )
    def body(idx_hbm, pidx_hbm, table_hbm, out_hbm, idx_v, pidx_v, rows_v, g_sem, o_sem):
        nc = plsc.get_sparse_core_info().num_cores
        wid = lax.axis_index("s") * nc + lax.axis_index("c")
        base = wid * n_per_w

        def start_gather(i, slot):
            off = base + i * _CHUNK
            pltpu.sync_copy(idx_hbm.at[pl.ds(off, _CHUNK)], idx_v.at[slot])
            pltpu.sync_copy(pidx_hbm.at[pl.ds(off, _CHUNK)], pidx_v.at[slot])
            pltpu.async_copy(
                table_hbm.at[idx_v.at[slot]], rows_v.at[slot], g_sem.at[slot]
            )

        def wait_gather(i, slot):
            pltpu.make_async_copy(
                table_hbm.at[idx_v.at[slot]], rows_v.at[slot], g_sem.at[slot]
            ).wait()

        def start_writeout(i, slot):
            pltpu.async_copy(
                rows_v.at[slot], out_hbm.at[pidx_v.at[slot]], o_sem.at[slot]
            )

        def wait_writeout(i, slot):
            pltpu.make_async_copy(
                rows_v.at[slot], out_hbm.at[pidx_v.at[slot]], o_sem.at[slot]
            ).wait()

        for i in range(_LOOKAHEAD):
            start_gather(i, i % _NBUF)

        def group(g, carry):
            for b in range(_NBUF):
                i = g * _NBUF + b
                j_slot = (b + _LOOKAHEAD) % _NBUF
                prev = i + _LOOKAHEAD - _NBUF

                @pl.when(prev >= 0)
                def _():
                    wait_writeout(prev, j_slot)

                @pl.when(i + _LOOKAHEAD < n_chunks)
                def _():
                    start_gather(i + _LOOKAHEAD, j_slot)

                wait_gather(i, b)
                start_writeout(i, b)
            return carry

        lax.fori_loop(0, n_groups, group, 0)

        for k in range(_NBUF - _LOOKAHEAD):
            i = n_chunks - (_NBUF - _LOOKAHEAD) + k
            wait_writeout(i, i % _NBUF)

    return body


def kernel(indices, vectors):
    B, S = indices.shape
    V, D = vectors.shape
    N = B * S
    info = plsc.get_sparse_core_info()
    n_workers = info.num_cores * info.num_subcores
    table2 = _pad_table(vectors).reshape(2 * V, D)
    flat_idx = 2 * indices.reshape(N).astype(jnp.int32)
    n = jnp.arange(N, dtype=jnp.int32)
    pidx = 2 * ((n // S) * _SEQ_PAD + n % S)
    out2 = _gather_kernel(N, 2 * V, n_workers)(flat_idx, pidx, table2)
    return out2.reshape(B, _SEQ_PAD, 128)[:, :S, :D]


# speedup vs baseline: 1.3004x; 1.3004x over previous
import functools

import jax
import jax.numpy as jnp
from jax import lax
from jax.experimental import pallas as pl
from jax.experimental.pallas import tpu as pltpu
from jax.experimental.pallas import tpu_sc as plsc

_NBUF = 4
_LOOKAHEAD = 2
_CHUNK = 400
_SEQ_PAD = 56
_PAD_BLK = 800


def _pad_table(vectors):
    V, D = vectors.shape

    def body(i_ref, o_ref):
        o_ref[:, :D] = i_ref[...]
        o_ref[:, D:] = jnp.zeros((_PAD_BLK, 128 - D), jnp.float32)

    return pl.pallas_call(
        body,
        grid=(V // _PAD_BLK,),
        in_specs=[pl.BlockSpec((_PAD_BLK, D), lambda i: (i, 0))],
        out_specs=pl.BlockSpec((_PAD_BLK, 128), lambda i: (i, 0)),
        out_shape=jax.ShapeDtypeStruct((V, 128), jnp.float32),
    )(vectors)


def _gather_kernel(N, V2, n_workers):
    n_per_w = N // n_workers
    n_chunks = n_per_w // _CHUNK
    n_groups = n_chunks // _NBUF
    mesh = plsc.VectorSubcoreMesh(core_axis_name="c", subcore_axis_name="s")

    @functools.partial(
        pl.kernel,
        mesh=mesh,
        out_type=jax.ShapeDtypeStruct((2 * 16384 * _SEQ_PAD, 64), jnp.float32),
        scratch_types=[
            pltpu.VMEM((_NBUF, _CHUNK), jnp.int32),
            pltpu.VMEM((_NBUF, _CHUNK), jnp.int32),
            pltpu.VMEM((_NBUF, _CHUNK, 64), jnp.float32),
            pltpu.SemaphoreType.DMA((_NBUF,)),
            pltpu.SemaphoreType.DMA((_NBUF,)),
        ],
        compiler_params=pltpu.CompilerParams(use_tc_tiling_on_sc=False),
    )
    def body(idx_hbm, pidx_hbm, table_hbm, out_hbm, idx_v, pidx_v, rows_v, g_sem, o_sem):
        nc = plsc.get_sparse_core_info().num_cores
        wid = lax.axis_index("s") * nc + lax.axis_index("c")
        base = wid * n_per_w

        def start_gather(i, slot):
            off = base + i * _CHUNK
            pltpu.sync_copy(idx_hbm.at[pl.ds(off, _CHUNK)], idx_v.at[slot])
            pltpu.sync_copy(pidx_hbm.at[pl.ds(off, _CHUNK)], pidx_v.at[slot])
            pltpu.async_copy(
                table_hbm.at[idx_v.at[slot]], rows_v.at[slot], g_sem.at[slot]
            )

        def wait_gather(i, slot):
            pltpu.make_async_copy(
                table_hbm.at[idx_v.at[slot]], rows_v.at[slot], g_sem.at[slot]
            ).wait()

        def start_writeout(i, slot):
            pltpu.async_copy(
                rows_v.at[slot], out_hbm.at[pidx_v.at[slot]], o_sem.at[slot]
            )

        def wait_writeout(i, slot):
            pltpu.make_async_copy(
                rows_v.at[slot], out_hbm.at[pidx_v.at[slot]], o_sem.at[slot]
            ).wait()

        for i in range(_LOOKAHEAD):
            start_gather(i, i % _NBUF)

        def group(g, carry):
            for b in range(_NBUF):
                i = g * _NBUF + b
                j_slot = (b + _LOOKAHEAD) % _NBUF
                prev = i + _LOOKAHEAD - _NBUF

                @pl.when(prev >= 0)
                def _():
                    wait_writeout(prev, j_slot)

                @pl.when(i + _LOOKAHEAD < n_chunks)
                def _():
                    start_gather(i + _LOOKAHEAD, j_slot)

                wait_gather(i, b)
                start_writeout(i, b)
            return carry

        lax.fori_loop(0, n_groups, group, 0)

        for k in range(_NBUF - _LOOKAHEAD):
            i = n_chunks - (_NBUF - _LOOKAHEAD) + k
            wait_writeout(i, i % _NBUF)

    return body


def kernel(indices, vectors):
    B, S = indices.shape
    V, D = vectors.shape
    N = B * S
    info = plsc.get_sparse_core_info()
    n_workers = info.num_cores * info.num_subcores
    table2 = jnp.concatenate(
        [vectors, jnp.zeros((V, 128 - D), jnp.float32)], axis=1
    ).reshape(2 * V, D)
    flat_idx = 2 * indices.reshape(N).astype(jnp.int32)
    n = jnp.arange(N, dtype=jnp.int32)
    pidx = 2 * ((n // S) * _SEQ_PAD + n % S)
    out2 = _gather_kernel(N, 2 * V, n_workers)(flat_idx, pidx, table2)
    return out2.reshape(B, _SEQ_PAD, 128)[:, :S, :D]


# speedup vs baseline: 1.5169x; 1.1665x over previous
import functools

import jax
import jax.numpy as jnp
from jax import lax
from jax.experimental import pallas as pl
from jax.experimental.pallas import tpu as pltpu
from jax.experimental.pallas import tpu_sc as plsc

_NBUF = 4
_LOOKAHEAD = 2
_CHUNK = 400
_SEQ_PAD = 56
_PAD_BLK = 10000


def _pad_table(vectors):
    V, D = vectors.shape

    def body(i_ref, o_ref):
        o_ref[:, :D] = i_ref[...]
        o_ref[:, D:] = jnp.zeros((_PAD_BLK, 128 - D), jnp.float32)

    return pl.pallas_call(
        body,
        grid=(V // _PAD_BLK,),
        in_specs=[pl.BlockSpec((_PAD_BLK, D), lambda i: (i, 0))],
        out_specs=pl.BlockSpec((_PAD_BLK, 128), lambda i: (i, 0)),
        out_shape=jax.ShapeDtypeStruct((V, 128), jnp.float32),
    )(vectors)


def _gather_kernel(N, V2, n_workers):
    n_per_w = N // n_workers
    n_chunks = n_per_w // _CHUNK
    n_groups = n_chunks // _NBUF
    mesh = plsc.VectorSubcoreMesh(core_axis_name="c", subcore_axis_name="s")

    @functools.partial(
        pl.kernel,
        mesh=mesh,
        out_type=jax.ShapeDtypeStruct((2 * 16384 * _SEQ_PAD, 64), jnp.float32),
        scratch_types=[
            pltpu.VMEM((_NBUF, _CHUNK), jnp.int32),
            pltpu.VMEM((_NBUF, _CHUNK), jnp.int32),
            pltpu.VMEM((_NBUF, _CHUNK, 64), jnp.float32),
            pltpu.SemaphoreType.DMA((_NBUF,)),
            pltpu.SemaphoreType.DMA((_NBUF,)),
        ],
        compiler_params=pltpu.CompilerParams(use_tc_tiling_on_sc=False),
    )
    def body(idx_hbm, pidx_hbm, table_hbm, out_hbm, idx_v, pidx_v, rows_v, g_sem, o_sem):
        nc = plsc.get_sparse_core_info().num_cores
        wid = lax.axis_index("s") * nc + lax.axis_index("c")
        base = wid * n_per_w

        def start_gather(i, slot):
            off = base + i * _CHUNK
            pltpu.sync_copy(idx_hbm.at[pl.ds(off, _CHUNK)], idx_v.at[slot])
            pltpu.sync_copy(pidx_hbm.at[pl.ds(off, _CHUNK)], pidx_v.at[slot])
            pltpu.async_copy(
                table_hbm.at[idx_v.at[slot]], rows_v.at[slot], g_sem.at[slot]
            )

        def wait_gather(i, slot):
            pltpu.make_async_copy(
                table_hbm.at[idx_v.at[slot]], rows_v.at[slot], g_sem.at[slot]
            ).wait()

        def start_writeout(i, slot):
            pltpu.async_copy(
                rows_v.at[slot], out_hbm.at[pidx_v.at[slot]], o_sem.at[slot]
            )

        def wait_writeout(i, slot):
            pltpu.make_async_copy(
                rows_v.at[slot], out_hbm.at[pidx_v.at[slot]], o_sem.at[slot]
            ).wait()

        for i in range(_LOOKAHEAD):
            start_gather(i, i % _NBUF)

        def group(g, carry):
            for b in range(_NBUF):
                i = g * _NBUF + b
                j_slot = (b + _LOOKAHEAD) % _NBUF
                prev = i + _LOOKAHEAD - _NBUF

                @pl.when(prev >= 0)
                def _():
                    wait_writeout(prev, j_slot)

                @pl.when(i + _LOOKAHEAD < n_chunks)
                def _():
                    start_gather(i + _LOOKAHEAD, j_slot)

                wait_gather(i, b)
                start_writeout(i, b)
            return carry

        lax.fori_loop(0, n_groups, group, 0)

        for k in range(_NBUF - _LOOKAHEAD):
            i = n_chunks - (_NBUF - _LOOKAHEAD) + k
            wait_writeout(i, i % _NBUF)

    return body


def kernel(indices, vectors):
    B, S = indices.shape
    V, D = vectors.shape
    N = B * S
    info = plsc.get_sparse_core_info()
    n_workers = info.num_cores * info.num_subcores
    table2 = _pad_table(vectors).reshape(2 * V, D)
    flat_idx = 2 * indices.reshape(N).astype(jnp.int32)
    n = jnp.arange(N, dtype=jnp.int32)
    pidx = 2 * ((n // S) * _SEQ_PAD + n % S)
    out2 = _gather_kernel(N, 2 * V, n_workers)(flat_idx, pidx, table2)
    return out2.reshape(B, _SEQ_PAD, 128)[:, :S, :D]
